# baseline (speedup 1.0000x reference)
import functools

import jax
import jax.numpy as jnp
from jax import lax
from jax.experimental import pallas as pl
from jax.experimental.pallas import tpu as pltpu
from jax.experimental.pallas import tpu_sc as plsc

NUM_CORES = 2
NUM_SUBCORES = 16
NUM_WORKERS = NUM_CORES * NUM_SUBCORES
LANES = 16


def _build(batch, maxlen, vocab, embed):
    assert batch % NUM_WORKERS == 0
    seq_per_w = batch // NUM_WORKERS
    mesh = plsc.VectorSubcoreMesh(
        core_axis_name="c", subcore_axis_name="s")

    gather_slices = []
    off = 0
    while off < maxlen:
        n = min(128, maxlen - off)
        gather_slices.append((off, n))
        off += n

    @functools.partial(
        pl.kernel,
        out_type=jax.ShapeDtypeStruct((batch * maxlen, embed), jnp.float32),
        mesh=mesh,
        scratch_types=[
            pltpu.VMEM((maxlen,), jnp.int32),
            pltpu.VMEM((maxlen, embed), jnp.float32),
            pltpu.VMEM((maxlen, embed), jnp.float32),
            pltpu.SemaphoreType.DMA,
        ],
        compiler_params=pltpu.CompilerParams(use_tc_tiling_on_sc=False),
    )
    def k(x_hbm, tok_hbm, pos_hbm, out_hbm, idx_v, buf_v, pos_v, sem):
        cid = lax.axis_index("c")
        sid = lax.axis_index("s")
        wid = sid * NUM_CORES + cid
        pltpu.sync_copy(pos_hbm, pos_v)

        @pl.loop(0, seq_per_w)
        def _seq(g):
            base = (wid * seq_per_w + g) * maxlen
            pltpu.sync_copy(x_hbm.at[pl.ds(base, maxlen)], idx_v)
            cps = []
            for off, n in gather_slices:
                cps.append(
                    pltpu.async_copy(
                        tok_hbm.at[idx_v.at[pl.ds(off, n)]],
                        buf_v.at[pl.ds(off, n)],
                        sem,
                    )
                )
            for cp in cps:
                cp.wait()

            @pl.loop(0, maxlen)
            def _row(r):
                for cstart in range(0, embed, LANES):
                    sl = pl.ds(cstart, LANES)
                    buf_v[r, sl] = buf_v[r, sl] + pos_v[r, sl]

            pltpu.sync_copy(buf_v, out_hbm.at[pl.ds(base, maxlen)])

    return k


def kernel(x, token_table, pos_table):
    batch, maxlen = x.shape
    vocab, embed = token_table.shape
    x_flat = x.reshape(batch * maxlen).astype(jnp.int32)
    k = _build(batch, maxlen, vocab, embed)
    out = k(x_flat, token_table, pos_table)
    return out.reshape(batch, maxlen, embed)
